# Initial kernel scaffold; baseline (speedup 1.0000x reference)
import jax
import jax.numpy as jnp
from jax.experimental import pallas as pl


def kernel(x, edge_index, edge_weight, W1, b1, W2, b2):
    raise NotImplementedError("write your pallas kernel here")


# speedup vs baseline: 3.3101x; 3.3101x over previous
import functools

import jax
import jax.numpy as jnp
from jax import lax
from jax.experimental import pallas as pl
from jax.experimental.pallas import tpu as pltpu
from jax.experimental.pallas import tpu_sc as plsc

_N = 10000
_E = 320000
_IN = 128
_HID = 128
_OUT = 64

_NC = 2
_NS = 16
_L = 16
_NW = _NC * _NS


_BM = 400


def _mm1_body(x_ref, w_ref, b_ref, o_ref):
    o_ref[...] = jnp.dot(x_ref[...], w_ref[...],
                         preferred_element_type=jnp.float32) + b_ref[...]


def _linear1(x, W1, b1):
    return pl.pallas_call(
        _mm1_body,
        grid=(_N // _BM,),
        in_specs=[pl.BlockSpec((_BM, _IN), lambda i: (i, 0)),
                  pl.BlockSpec((_IN, _HID), lambda i: (0, 0)),
                  pl.BlockSpec((1, _HID), lambda i: (0, 0))],
        out_specs=pl.BlockSpec((_BM, _HID), lambda i: (i, 0)),
        out_shape=jax.ShapeDtypeStruct((_N, _HID), jnp.float32),
    )(x, W1, b1[None])


def _mm2_body(p0_ref, p1_ref, w_ref, b_ref, o_ref):
    h = jnp.maximum(p0_ref[...] + p1_ref[...], 0.0)
    o_ref[...] = jnp.dot(h, w_ref[...],
                         preferred_element_type=jnp.float32) + b_ref[...]


def _linear2(p0, p1, W2, b2):
    return pl.pallas_call(
        _mm2_body,
        grid=(_N // _BM,),
        in_specs=[pl.BlockSpec((_BM, _HID), lambda i: (i, 0)),
                  pl.BlockSpec((_BM, _HID), lambda i: (i, 0)),
                  pl.BlockSpec((_HID, _OUT), lambda i: (0, 0)),
                  pl.BlockSpec((1, _OUT), lambda i: (0, 0))],
        out_specs=pl.BlockSpec((_BM, _OUT), lambda i: (i, 0)),
        out_shape=jax.ShapeDtypeStruct((_N, _OUT), jnp.float32),
    )(p0, p1, W2, b2[None])


def _add_body(a_ref, b_ref, o_ref):
    o_ref[...] = a_ref[...] + b_ref[...]


def _final_add(a, b):
    m, n = a.shape
    bm = m // 25
    return pl.pallas_call(
        _add_body,
        grid=(25,),
        in_specs=[pl.BlockSpec((bm, n), lambda i: (i, 0)),
                  pl.BlockSpec((bm, n), lambda i: (i, 0))],
        out_specs=pl.BlockSpec((bm, n), lambda i: (i, 0)),
        out_shape=jax.ShapeDtypeStruct((m, n), jnp.float32),
    )(a, b)


def _make_spmm(D):
    EW = _E // _NW
    CH = 80
    NCH = EW // CH
    RPT = 632
    RPT_LAST = _N - (_NS - 1) * RPT
    NF = D // _L

    mesh = plsc.VectorSubcoreMesh(core_axis_name="c", subcore_axis_name="s",
                                  num_cores=_NC)

    @functools.partial(
        pl.kernel, mesh=mesh,
        compiler_params=pltpu.CompilerParams(use_tc_tiling_on_sc=False),
        out_type=jax.ShapeDtypeStruct((_NC, _N, D), jnp.float32),
        scratch_types=[
            pltpu.VMEM_SHARED((_N, D), jnp.float32),
            pltpu.VMEM((CH,), jnp.int32),
            pltpu.VMEM((CH,), jnp.int32),
            pltpu.VMEM((CH,), jnp.float32),
            pltpu.VMEM((CH, D), jnp.float32),
            pltpu.SemaphoreType.DMA,
        ])
    def spmm(h_hbm, src_hbm, dst_hbm, w_hbm, out_hbm,
             acc, src_v, dst_v, w_v, rows_v, sem):
        c = lax.axis_index("c")
        s = lax.axis_index("s")
        wid = s * _NC + c
        ebase = wid * EW
        rbase = s * RPT

        zero = jnp.zeros((_L,), jnp.float32)

        def zrow(i, carry):
            for j in range(NF):
                rows_v[i, pl.ds(j * _L, _L)] = zero
            return carry
        lax.fori_loop(0, CH, zrow, 0)

        def zero_acc(nrows):
            nfull, tail = divmod(nrows, CH)

            def f():
                for k in range(nfull):
                    pltpu.sync_copy(rows_v, acc.at[pl.ds(rbase + k * CH, CH)])
                if tail:
                    pltpu.sync_copy(rows_v.at[pl.ds(0, tail)],
                                    acc.at[pl.ds(rbase + nfull * CH, tail)])
            return f
        pl.when(s < _NS - 1)(zero_acc(RPT))
        pl.when(s == _NS - 1)(zero_acc(RPT_LAST))
        plsc.subcore_barrier()

        def chunk(i, carry):
            base = ebase + i * CH
            pltpu.sync_copy(src_hbm.at[pl.ds(base, CH)], src_v)
            pltpu.sync_copy(dst_hbm.at[pl.ds(base, CH)], dst_v)
            pltpu.sync_copy(w_hbm.at[pl.ds(base, CH)], w_v)
            pltpu.async_copy(h_hbm.at[src_v], rows_v, sem).wait()

            def scale(g, c2):
                wv16 = w_v[pl.ds(g * _L, _L)]
                for l in range(_L):
                    e = g * _L + l
                    wv = jnp.full((_L,), wv16[l], jnp.float32)
                    for j in range(NF):
                        sl = pl.ds(j * _L, _L)
                        rows_v[e, sl] = rows_v[e, sl] * wv
                return c2
            lax.fori_loop(0, CH // _L, scale, 0)

            pltpu.sync_copy(rows_v, acc.at[dst_v], add=True)
            return carry
        lax.fori_loop(0, NCH, chunk, 0)

        plsc.subcore_barrier()

        def writeback(nrows):
            def f():
                pltpu.sync_copy(acc.at[pl.ds(rbase, nrows)],
                                out_hbm.at[c, pl.ds(rbase, nrows)])
            return f
        pl.when(s < _NS - 1)(writeback(RPT))
        pl.when(s == _NS - 1)(writeback(RPT_LAST))

    return spmm


_spmm_cache = {}


def _spmm(D):
    if D not in _spmm_cache:
        _spmm_cache[D] = _make_spmm(D)
    return _spmm_cache[D]


def kernel(x, edge_index, edge_weight, W1, b1, W2, b2):
    src = edge_index[0].astype(jnp.int32)
    dst = edge_index[1].astype(jnp.int32)
    w = edge_weight.astype(jnp.float32)

    h1 = _linear1(x, W1, b1)
    p = _spmm(_HID)(h1, src, dst, w)
    h2 = _linear2(p[0], p[1], W2, b2)
    q = _spmm(_OUT)(h2, src, dst, w)
    out = _final_add(q[0].reshape(_N // 2, 2 * _OUT),
                     q[1].reshape(_N // 2, 2 * _OUT))
    return out.reshape(_N, _OUT)


# speedup vs baseline: 7.4336x; 2.2457x over previous
import functools

import jax
import jax.numpy as jnp
from jax import lax
from jax.experimental import pallas as pl
from jax.experimental.pallas import tpu as pltpu
from jax.experimental.pallas import tpu_sc as plsc

_N = 10000
_E = 320000
_IN = 128
_HID = 128
_OUT = 64

_NC = 2
_NS = 16
_L = 16
_NW = _NC * _NS


_BM = 400


def _mm1_body(x_ref, w_ref, b_ref, o_ref):
    o_ref[...] = jnp.dot(x_ref[...], w_ref[...],
                         preferred_element_type=jnp.float32) + b_ref[...]


def _linear1(x, W1, b1):
    return pl.pallas_call(
        _mm1_body,
        grid=(_N // _BM,),
        in_specs=[pl.BlockSpec((_BM, _IN), lambda i: (i, 0)),
                  pl.BlockSpec((_IN, _HID), lambda i: (0, 0)),
                  pl.BlockSpec((1, _HID), lambda i: (0, 0))],
        out_specs=pl.BlockSpec((_BM, _HID), lambda i: (i, 0)),
        out_shape=jax.ShapeDtypeStruct((_N, _HID), jnp.float32),
    )(x, W1, b1[None])


def _mm2_body(p0_ref, p1_ref, w_ref, b_ref, o_ref):
    h = jnp.maximum(p0_ref[...] + p1_ref[...], 0.0)
    o_ref[...] = jnp.dot(h, w_ref[...],
                         preferred_element_type=jnp.float32) + b_ref[...]


def _linear2(p0, p1, W2, b2):
    return pl.pallas_call(
        _mm2_body,
        grid=(_N // _BM,),
        in_specs=[pl.BlockSpec((_BM, _HID), lambda i: (i, 0)),
                  pl.BlockSpec((_BM, _HID), lambda i: (i, 0)),
                  pl.BlockSpec((_HID, _OUT), lambda i: (0, 0)),
                  pl.BlockSpec((1, _OUT), lambda i: (0, 0))],
        out_specs=pl.BlockSpec((_BM, _OUT), lambda i: (i, 0)),
        out_shape=jax.ShapeDtypeStruct((_N, _OUT), jnp.float32),
    )(p0, p1, W2, b2[None])


def _add_body(a_ref, b_ref, o_ref):
    o_ref[...] = a_ref[...] + b_ref[...]


def _final_add(a, b):
    m, n = a.shape
    bm = m // 25
    return pl.pallas_call(
        _add_body,
        grid=(25,),
        in_specs=[pl.BlockSpec((bm, n), lambda i: (i, 0)),
                  pl.BlockSpec((bm, n), lambda i: (i, 0))],
        out_specs=pl.BlockSpec((bm, n), lambda i: (i, 0)),
        out_shape=jax.ShapeDtypeStruct((m, n), jnp.float32),
    )(a, b)


def _make_spmm(D):
    EW = _E // _NW
    CH = 80
    NCH = EW // CH
    G = 25
    NSUP = NCH // G
    NBUF = 3
    LA = 2
    RPT = 632
    RPT_LAST = _N - (_NS - 1) * RPT
    NF = D // _L

    mesh = plsc.VectorSubcoreMesh(core_axis_name="c", subcore_axis_name="s",
                                  num_cores=_NC)

    @functools.partial(
        pl.kernel, mesh=mesh,
        compiler_params=pltpu.CompilerParams(use_tc_tiling_on_sc=False),
        out_type=jax.ShapeDtypeStruct((_NC, _N, D), jnp.float32),
        scratch_types=[
            pltpu.VMEM_SHARED((_N, D), jnp.float32),
            pltpu.VMEM((G, CH), jnp.int32),
            pltpu.VMEM((G, CH), jnp.int32),
            pltpu.VMEM((G, CH), jnp.float32),
        ] + [pltpu.VMEM((CH, D), jnp.float32)] * NBUF
          + [pltpu.SemaphoreType.DMA] * (2 * NBUF))
    def spmm(h_hbm, src_hbm, dst_hbm, w_hbm, out_hbm,
             acc, src_i, dst_i, w_i,
             r0, r1, r2, g0, g1, g2, s0, s1, s2):
        rows = (r0, r1, r2)
        gsem = (g0, g1, g2)
        ssem = (s0, s1, s2)
        c = lax.axis_index("c")
        s = lax.axis_index("s")
        wid = s * _NC + c
        rbase = s * RPT

        zero = jnp.zeros((_L,), jnp.float32)

        def zrow(i, carry):
            for j in range(NF):
                r0[i, pl.ds(j * _L, _L)] = zero
            return carry
        lax.fori_loop(0, CH, zrow, 0)

        def zero_acc(nrows):
            nfull, tail = divmod(nrows, CH)

            def f():
                for k in range(nfull):
                    pltpu.sync_copy(r0, acc.at[pl.ds(rbase + k * CH, CH)])
                if tail:
                    pltpu.sync_copy(r0.at[pl.ds(0, tail)],
                                    acc.at[pl.ds(rbase + nfull * CH, tail)])
            return f
        pl.when(s < _NS - 1)(zero_acc(RPT))
        pl.when(s == _NS - 1)(zero_acc(RPT_LAST))
        plsc.subcore_barrier()

        def fire_gather(m, b):
            pltpu.async_copy(h_hbm.at[src_i.at[m]], rows[b], gsem[b])

        def wait_gather(m, b):
            pltpu.make_async_copy(h_hbm.at[src_i.at[m]], rows[b],
                                  gsem[b]).wait()

        def fire_scatter(m, b):
            pltpu.async_copy(rows[b], acc.at[dst_i.at[m]], ssem[b], add=True)

        def wait_scatter(m, b):
            pltpu.make_async_copy(rows[b], acc.at[dst_i.at[m]],
                                  ssem[b]).wait()

        def scale(m, b):
            buf = rows[b]

            def grp(g, c2):
                wv16 = w_i[m, pl.ds(g * _L, _L)]
                for l in range(_L):
                    e = g * _L + l
                    wv = jnp.full((_L,), wv16[l], jnp.float32)
                    for j in range(NF):
                        sl = pl.ds(j * _L, _L)
                        buf[e, sl] = buf[e, sl] * wv
                return c2
            lax.fori_loop(0, CH // _L, grp, 0)

        def slot(m, b, wait_prev_scatter, gather_ahead):
            wait_gather(m, b)
            scale(m, b)
            fire_scatter(m, b)
            b2 = (b + LA) % NBUF
            if gather_ahead:
                if wait_prev_scatter:
                    wait_scatter(m - (NBUF - LA), b2)
                fire_gather(m + LA, b2)

        def super_body(u, carry):
            pltpu.sync_copy(src_hbm.at[wid, pl.ds(u * G, G)], src_i)
            pltpu.sync_copy(dst_hbm.at[wid, pl.ds(u * G, G)], dst_i)
            pltpu.sync_copy(w_hbm.at[wid, pl.ds(u * G, G)], w_i)

            fire_gather(0, 0)
            fire_gather(1, 1)
            slot(0, 0, False, True)
            slot(1, 1, True, True)

            def round_body(g, c2):
                m0 = 2 + g * NBUF
                for k in range(NBUF):
                    slot(m0 + k, (2 + k) % NBUF, True, True)
                return c2
            lax.fori_loop(0, (G - 4) // NBUF, round_body, 0)

            slot(G - 2, (G - 2) % NBUF, False, False)
            slot(G - 1, (G - 1) % NBUF, False, False)
            for m in range(G - NBUF, G):
                wait_scatter(m, m % NBUF)
            return carry
        lax.fori_loop(0, NSUP, super_body, 0)

        plsc.subcore_barrier()

        def writeback(nrows):
            def f():
                pltpu.sync_copy(acc.at[pl.ds(rbase, nrows)],
                                out_hbm.at[c, pl.ds(rbase, nrows)])
            return f
        pl.when(s < _NS - 1)(writeback(RPT))
        pl.when(s == _NS - 1)(writeback(RPT_LAST))

    return spmm


_spmm_cache = {}


def _spmm(D):
    if D not in _spmm_cache:
        _spmm_cache[D] = _make_spmm(D)
    return _spmm_cache[D]


def kernel(x, edge_index, edge_weight, W1, b1, W2, b2):
    ew_ = _E // _NW
    ch = 80
    src = edge_index[0].astype(jnp.int32).reshape(_NW, ew_ // ch, ch)
    dst = edge_index[1].astype(jnp.int32).reshape(_NW, ew_ // ch, ch)
    w = edge_weight.astype(jnp.float32).reshape(_NW, ew_ // ch, ch)

    h1 = _linear1(x, W1, b1)
    p = _spmm(_HID)(h1, src, dst, w)
    h2 = _linear2(p[0], p[1], W2, b2)
    q = _spmm(_OUT)(h2, src, dst, w)
    out = _final_add(q[0].reshape(_N // 2, 2 * _OUT),
                     q[1].reshape(_N // 2, 2 * _OUT))
    return out.reshape(_N, _OUT)
